# Initial kernel scaffold; baseline (speedup 1.0000x reference)
import jax
import jax.numpy as jnp
from jax.experimental import pallas as pl


def kernel(time, table, W, b):
    raise NotImplementedError("write your pallas kernel here")


# speedup vs baseline: 27.3021x; 27.3021x over previous
import functools

import jax
import jax.numpy as jnp
from jax import lax
from jax.experimental import pallas as pl
from jax.experimental.pallas import tpu as pltpu
from jax.experimental.pallas import tpu_sc as plsc

HIDDEN = 128
TWO_H = 256
ROWS_PER_BLOCK = 1000


def _project_kernel(t_ref, w_ref, b_ref, o_ref):
    o_ref[...] = (
        lax.dot_general(
            t_ref[...], w_ref[...],
            (((1,), (1,)), ((), ())),
            preferred_element_type=jnp.float32,
        )
        + b_ref[...]
    )


def _project_table(table, W, b):
    n_rows = table.shape[0]
    grid = (n_rows // ROWS_PER_BLOCK,)
    return pl.pallas_call(
        _project_kernel,
        grid=grid,
        in_specs=[
            pl.BlockSpec((ROWS_PER_BLOCK, TWO_H), lambda i: (i, 0)),
            pl.BlockSpec((HIDDEN, TWO_H), lambda i: (0, 0)),
            pl.BlockSpec((1, HIDDEN), lambda i: (0, 0)),
        ],
        out_specs=pl.BlockSpec((ROWS_PER_BLOCK, HIDDEN), lambda i: (i, 0)),
        out_shape=jax.ShapeDtypeStruct((n_rows, HIDDEN), jnp.float32),
    )(table, W, b.reshape(1, HIDDEN))


CHUNK = 128


def _make_gather(n_idx):
    info = plsc.get_sparse_core_info()
    nw = info.num_cores * info.num_subcores
    assert n_idx % (nw * CHUNK) == 0
    chunks_per_w = n_idx // (nw * CHUNK)
    mesh = plsc.VectorSubcoreMesh(core_axis_name="c", subcore_axis_name="s")

    @functools.partial(
        pl.kernel,
        mesh=mesh,
        out_type=jax.ShapeDtypeStruct((n_idx, HIDDEN), jnp.float32),
        scratch_types=[
            pltpu.VMEM((chunks_per_w, CHUNK), jnp.int32),
            pltpu.VMEM((CHUNK, HIDDEN), jnp.float32),
            pltpu.SemaphoreType.DMA,
        ],
    )
    def gather_k(ptable_hbm, idx_hbm, out_hbm, idx_v, rows_v, sem):
        wid = lax.axis_index("s") * info.num_cores + lax.axis_index("c")
        base = wid * chunks_per_w
        pltpu.sync_copy(idx_hbm.at[pl.ds(base, chunks_per_w)], idx_v)

        def body(j, carry):
            start = (base + j) * CHUNK
            pltpu.async_copy(ptable_hbm.at[idx_v.at[j]], rows_v, sem).wait()
            pltpu.sync_copy(rows_v, out_hbm.at[pl.ds(start, CHUNK)])
            return carry

        lax.fori_loop(0, chunks_per_w, body, 0)

    return gather_k


def kernel(time, table, W, b):
    B, L = time.shape
    n_idx = B * L
    ptable = _project_table(table, W, b)
    idx2d = time.astype(jnp.int32).reshape(n_idx // CHUNK, CHUNK)
    out = _make_gather(n_idx)(ptable, idx2d)
    return out.reshape(B, L, HIDDEN)


# speedup vs baseline: 36.6797x; 1.3435x over previous
import functools

import jax
import jax.numpy as jnp
from jax import lax
from jax.experimental import pallas as pl
from jax.experimental.pallas import tpu as pltpu
from jax.experimental.pallas import tpu_sc as plsc

HIDDEN = 128
TWO_H = 256
ROWS_PER_BLOCK = 1000


def _project_kernel(t_ref, w_ref, b_ref, o_ref):
    o_ref[...] = (
        lax.dot_general(
            t_ref[...], w_ref[...],
            (((1,), (1,)), ((), ())),
            preferred_element_type=jnp.float32,
        )
        + b_ref[...]
    )


def _project_table(table, W, b):
    n_rows = table.shape[0]
    grid = (n_rows // ROWS_PER_BLOCK,)
    return pl.pallas_call(
        _project_kernel,
        grid=grid,
        in_specs=[
            pl.BlockSpec((ROWS_PER_BLOCK, TWO_H), lambda i: (i, 0)),
            pl.BlockSpec((HIDDEN, TWO_H), lambda i: (0, 0)),
            pl.BlockSpec((1, HIDDEN), lambda i: (0, 0)),
        ],
        out_specs=pl.BlockSpec((ROWS_PER_BLOCK, HIDDEN), lambda i: (i, 0)),
        out_shape=jax.ShapeDtypeStruct((n_rows, HIDDEN), jnp.float32),
    )(table, W, b.reshape(1, HIDDEN))


CHUNK = 128
CPG = 2


def _make_gather(n_idx):
    info = plsc.get_sparse_core_info()
    nw = info.num_cores * info.num_subcores
    assert n_idx % (nw * CHUNK * CPG * 2) == 0
    chunks_per_w = n_idx // (nw * CHUNK)
    n_groups = chunks_per_w // CPG
    n_pairs = n_groups // 2
    grows = CPG * CHUNK
    mesh = plsc.VectorSubcoreMesh(core_axis_name="c", subcore_axis_name="s")

    @functools.partial(
        pl.kernel,
        mesh=mesh,
        out_type=jax.ShapeDtypeStruct((n_idx, HIDDEN), jnp.float32),
        scratch_types=[
            pltpu.VMEM((chunks_per_w, CHUNK), jnp.int32),
            pltpu.VMEM((grows, HIDDEN), jnp.float32),
            pltpu.VMEM((grows, HIDDEN), jnp.float32),
            pltpu.SemaphoreType.DMA,
            pltpu.SemaphoreType.DMA,
        ],
    )
    def gather_k(ptable_hbm, idx_hbm, out_hbm, idx_v, rows_a, rows_b, sem_a, sem_b):
        wid = lax.axis_index("s") * info.num_cores + lax.axis_index("c")
        base = wid * chunks_per_w
        pltpu.sync_copy(idx_hbm.at[pl.ds(base, chunks_per_w)], idx_v)

        bufs = (rows_a, rows_b)
        sems = (sem_a, sem_b)

        def issue(g, slot):
            gg = jnp.minimum(g, n_groups - 1)
            for c in range(CPG):
                pltpu.async_copy(
                    ptable_hbm.at[idx_v.at[gg * CPG + c]],
                    bufs[slot].at[pl.ds(c * CHUNK, CHUNK)],
                    sems[slot],
                )

        def drain(slot):
            pltpu.make_async_copy(
                ptable_hbm.at[pl.ds(0, grows)], bufs[slot], sems[slot]
            ).wait()

        def store(g, slot):
            pltpu.sync_copy(
                bufs[slot], out_hbm.at[pl.ds((base + g * CPG) * CHUNK, grows)]
            )

        issue(0, 0)
        issue(1, 1)

        def pair_body(p, carry):
            g0 = 2 * p
            drain(0)
            store(g0, 0)
            issue(g0 + 2, 0)
            drain(1)
            store(g0 + 1, 1)
            issue(g0 + 3, 1)
            return carry

        lax.fori_loop(0, n_pairs, pair_body, 0)
        drain(0)
        drain(1)

    return gather_k


def kernel(time, table, W, b):
    B, L = time.shape
    n_idx = B * L
    ptable = _project_table(table, W, b)
    idx2d = time.astype(jnp.int32).reshape(n_idx // CHUNK, CHUNK)
    out = _make_gather(n_idx)(ptable, idx2d)
    return out.reshape(B, L, HIDDEN)
